# baseline (speedup 1.0000x reference)
import functools

import jax
import jax.numpy as jnp
from jax import lax
from jax.experimental import pallas as pl
from jax.experimental.pallas import tpu as pltpu
from jax.experimental.pallas import tpu_sc as plsc

NUM_FEATURES = 128
NUM_EDGES = 160000
NUM_NODES = 10000
ROWS = NUM_NODES + NUM_EDGES
BATCH = 4
TOTAL_ROWS = BATCH * ROWS

NC, NS = 2, 16
N_IDX = 128

CHUNK = 200
CHUNKS_PER_BATCH = ROWS // CHUNK
K_MAX = -(-CHUNKS_PER_BATCH // NS)

_sc_mesh = plsc.VectorSubcoreMesh(core_axis_name="c", subcore_axis_name="s")


@functools.partial(
    pl.kernel,
    out_type=jax.ShapeDtypeStruct((TOTAL_ROWS, NUM_FEATURES), jnp.float32),
    mesh=_sc_mesh,
    scratch_types=[
        pltpu.VMEM((CHUNK, NUM_FEATURES), jnp.float32),
        pltpu.VMEM((N_IDX,), jnp.int32),
        pltpu.VMEM((N_IDX, NUM_FEATURES), jnp.float32),
        pltpu.SemaphoreType.DMA,
        pltpu.SemaphoreType.DMA,
    ],
)
def _sc_all(x_hbm, idx_hbm, out_ref, zbuf, idx_v, rows_v, sem_z, sem_s):
    c = lax.axis_index("c")
    s = lax.axis_index("s")
    is_scatterer = s >= NS - 2
    b_sc = jnp.where(s == NS - 1, c, c + 2)

    @pl.when(is_scatterer)
    def _():
        pltpu.async_copy(idx_hbm.at[0], idx_v, sem_s)
        pltpu.async_copy(x_hbm.at[pl.ds(b_sc * N_IDX, N_IDX)], rows_v, sem_s)

    z16 = jnp.zeros((16,), jnp.float32)

    @pl.loop(0, CHUNK)
    def _(i):
        for j in range(NUM_FEATURES // 16):
            zbuf[i, pl.ds(j * 16, 16)] = z16

    def chunk_dst(b2, k):
        batch = c + 2 * b2
        r = k * NS + s
        return r, out_ref.at[pl.ds(batch * ROWS + r * CHUNK, CHUNK)]

    for b2 in range(2):
        for k in range(K_MAX):
            r, dst = chunk_dst(b2, k)

            @pl.when(r < CHUNKS_PER_BATCH)
            def _():
                pltpu.async_copy(zbuf, dst, sem_z)

    for b2 in range(2):
        for k in range(K_MAX):
            r, dst = chunk_dst(b2, k)

            @pl.when(r < CHUNKS_PER_BATCH)
            def _():
                pltpu.make_async_copy(zbuf, dst, sem_z).wait()

    plsc.subcore_barrier()

    @pl.when(is_scatterer)
    def _():
        pltpu.make_async_copy(idx_hbm.at[0], idx_v, sem_s).wait()
        pltpu.make_async_copy(
            x_hbm.at[pl.ds(b_sc * N_IDX, N_IDX)], rows_v, sem_s
        ).wait()
        off = b_sc * ROWS + NUM_EDGES
        for j in range(N_IDX // 16):
            sl = pl.ds(j * 16, 16)
            idx_v[sl] = idx_v[sl] + off
        pltpu.sync_copy(rows_v, out_ref.at[idx_v])


def kernel(x, pivotal_nodes):
    bsz, n_idx, f = x.shape
    x_flat = x.reshape(bsz * n_idx, f)
    idx2 = pivotal_nodes.reshape(1, N_IDX)
    return _sc_all(x_flat, idx2).reshape(bsz, ROWS, f)


# speedup vs baseline: 1.0071x; 1.0071x over previous
import functools

import jax
import jax.numpy as jnp
from jax import lax
from jax.experimental import pallas as pl
from jax.experimental.pallas import tpu as pltpu
from jax.experimental.pallas import tpu_sc as plsc

NUM_FEATURES = 128
NUM_EDGES = 160000
NUM_NODES = 10000
ROWS = NUM_NODES + NUM_EDGES
BATCH = 4
TOTAL_ROWS = BATCH * ROWS

NC, NS = 2, 16
N_IDX = 128

CHUNK = 400
CHUNKS_PER_BATCH = ROWS // CHUNK
K_MAX = -(-CHUNKS_PER_BATCH // NS)

_sc_mesh = plsc.VectorSubcoreMesh(core_axis_name="c", subcore_axis_name="s")


@functools.partial(
    pl.kernel,
    out_type=jax.ShapeDtypeStruct((TOTAL_ROWS, NUM_FEATURES), jnp.float32),
    mesh=_sc_mesh,
    scratch_types=[
        pltpu.VMEM((CHUNK, NUM_FEATURES), jnp.float32),
        pltpu.VMEM((N_IDX,), jnp.int32),
        pltpu.VMEM((N_IDX, NUM_FEATURES), jnp.float32),
        pltpu.SemaphoreType.DMA,
        pltpu.SemaphoreType.DMA,
    ],
)
def _sc_all(x_hbm, idx_hbm, out_ref, zbuf, idx_v, rows_v, sem_z, sem_s):
    c = lax.axis_index("c")
    s = lax.axis_index("s")
    is_scatterer = s >= NS - 2
    b_sc = jnp.where(s == NS - 1, c, c + 2)

    @pl.when(is_scatterer)
    def _():
        pltpu.async_copy(idx_hbm.at[0], idx_v, sem_s)
        pltpu.async_copy(x_hbm.at[pl.ds(b_sc * N_IDX, N_IDX)], rows_v, sem_s)

    z16 = jnp.zeros((16,), jnp.float32)

    @pl.loop(0, CHUNK)
    def _(i):
        for j in range(NUM_FEATURES // 16):
            zbuf[i, pl.ds(j * 16, 16)] = z16

    def chunk_dst(b2, k):
        batch = c + 2 * b2
        r = k * NS + s
        return r, out_ref.at[pl.ds(batch * ROWS + r * CHUNK, CHUNK)]

    for b2 in range(2):
        for k in range(K_MAX):
            r, dst = chunk_dst(b2, k)

            @pl.when(r < CHUNKS_PER_BATCH)
            def _():
                pltpu.async_copy(zbuf, dst, sem_z)

    for b2 in range(2):
        for k in range(K_MAX):
            r, dst = chunk_dst(b2, k)

            @pl.when(r < CHUNKS_PER_BATCH)
            def _():
                pltpu.make_async_copy(zbuf, dst, sem_z).wait()

    plsc.subcore_barrier()

    @pl.when(is_scatterer)
    def _():
        pltpu.make_async_copy(idx_hbm.at[0], idx_v, sem_s).wait()
        pltpu.make_async_copy(
            x_hbm.at[pl.ds(b_sc * N_IDX, N_IDX)], rows_v, sem_s
        ).wait()
        off = b_sc * ROWS + NUM_EDGES
        for j in range(N_IDX // 16):
            sl = pl.ds(j * 16, 16)
            idx_v[sl] = idx_v[sl] + off
        pltpu.sync_copy(rows_v, out_ref.at[idx_v])


def kernel(x, pivotal_nodes):
    bsz, n_idx, f = x.shape
    x_flat = x.reshape(bsz * n_idx, f)
    idx2 = pivotal_nodes.reshape(1, N_IDX)
    return _sc_all(x_flat, idx2).reshape(bsz, ROWS, f)


# speedup vs baseline: 1.1846x; 1.1762x over previous
import jax
import jax.numpy as jnp
from jax.experimental import pallas as pl
from jax.experimental.pallas import tpu as pltpu

NUM_FEATURES = 128
NUM_EDGES = 160000
NUM_NODES = 10000
ROWS = NUM_NODES + NUM_EDGES
BATCH = 4
N_IDX = 128

ZCHUNK = 8000
N_Z = NUM_EDGES // ZCHUNK


def _body(idx_ref, x_ref, out_ref, zbuf, nbuf, sem_z, sem_n):
    zbuf[...] = jnp.zeros_like(zbuf)
    for b in range(BATCH):
        for k in range(N_Z):
            pltpu.make_async_copy(
                zbuf, out_ref.at[pl.ds(b * ROWS + k * ZCHUNK, ZCHUNK)], sem_z
            ).start()

    nbuf[...] = jnp.zeros_like(nbuf)

    def write_row(i, b):
        nbuf[b, pl.ds(idx_ref[i], 1), :] = x_ref[b, pl.ds(i, 1), :]
        return b

    for b in range(BATCH):
        jax.lax.fori_loop(0, N_IDX, write_row, b)
        pltpu.make_async_copy(
            nbuf.at[b], out_ref.at[pl.ds(b * ROWS + NUM_EDGES, NUM_NODES)], sem_n
        ).start()

    for b in range(BATCH):
        for k in range(N_Z):
            pltpu.make_async_copy(
                zbuf, out_ref.at[pl.ds(b * ROWS + k * ZCHUNK, ZCHUNK)], sem_z
            ).wait()
        pltpu.make_async_copy(
            nbuf.at[b], out_ref.at[pl.ds(b * ROWS + NUM_EDGES, NUM_NODES)], sem_n
        ).wait()


def kernel(x, pivotal_nodes):
    bsz, n_idx, f = x.shape
    grid_spec = pltpu.PrefetchScalarGridSpec(
        num_scalar_prefetch=1,
        grid=(1,),
        in_specs=[pl.BlockSpec((bsz, n_idx, f), lambda i, idx: (0, 0, 0))],
        out_specs=pl.BlockSpec(memory_space=pl.ANY),
        scratch_shapes=[
            pltpu.VMEM((ZCHUNK, f), jnp.float32),
            pltpu.VMEM((bsz, NUM_NODES, f), jnp.float32),
            pltpu.SemaphoreType.DMA,
            pltpu.SemaphoreType.DMA,
        ],
    )
    out_flat = pl.pallas_call(
        _body,
        grid_spec=grid_spec,
        out_shape=jax.ShapeDtypeStruct((bsz * ROWS, f), x.dtype),
    )(pivotal_nodes, x)
    return out_flat.reshape(bsz, ROWS, f)
